# baseline (speedup 1.0000x reference)
import functools

import numpy as np
import jax
import jax.numpy as jnp
from jax.experimental import pallas as pl

_B, _C, _W0, _H0 = 8, 192, 56, 56
_HEADS, _HD = 4, 24
_OC = _HEADS * _HD
_S = _W0 * _H0
_NJ = 16
_NPTS = _B * _HEADS * _S
_NORTH = _B * _HEADS * 4 * 16


def _constants():
    s = np.arange(_S)
    w, h = s // _H0, s % _H0
    j = (w // 28) * 8 + (h // 28) * 4 + ((w // 14) % 2) * 2 + ((h // 14) % 2)
    rows = np.arange(_NJ)[:, None]
    pool = (rows == j[None, :]).astype(np.float32) / 196.0
    valid = ((rows // 4) == (j[None, :] // 4)).astype(np.float32)
    ridx = np.arange(_NJ, dtype=np.float32)[:, None]
    eye = np.eye(_NJ, dtype=np.float32)
    blk = ((np.arange(_NJ)[:, None] // 4) ==
           (np.arange(_NJ)[None, :] // 4)).astype(np.float32)
    return pool, valid, ridx, eye, blk


_POOL, _VALID, _RIDX, _EYE, _BLK = _constants()


def _fec_kernel(x_ref, wfv_ref, bfv_ref, wp_ref, bp_ref,
                ab_ref, pool_ref, valid_ref, ridx_ref, eye_ref, blk_ref,
                out_ref, lc_ref, ls_ref, lo_ref):
    i = pl.program_id(0)

    @pl.when(i == 0)
    def _init():
        lc_ref[...] = jnp.zeros((1, 1), jnp.float32)
        ls_ref[...] = jnp.zeros((1, 1), jnp.float32)
        lo_ref[...] = jnp.zeros((1, 1), jnp.float32)

    f32 = jnp.float32
    bf16 = jnp.bfloat16
    dot = functools.partial(jax.lax.dot_general, preferred_element_type=f32)
    hdot = functools.partial(jax.lax.dot_general, preferred_element_type=f32,
                             precision=jax.lax.Precision.HIGHEST)
    xb = x_ref[0]
    fv = dot(wfv_ref[...].astype(bf16), xb,
             (((1,), (0,)), ((), ()))) + bfv_ref[...]
    feat, val = fv[:_OC], fv[_OC:]
    cents = hdot(fv, pool_ref[...], (((1,), (1,)), ((), ())))
    cent, vcent = cents[:_OC], cents[_OC:]
    alpha = ab_ref[0, 0]
    beta = ab_ref[0, 1]
    validm = valid_ref[...]
    ridx = ridx_ref[...]
    eps = 1e-12
    ones_row = jnp.ones((1, _S), f32)
    disp_parts = []
    lc_acc = 0.0
    ls_acc = 0.0
    lo_acc = 0.0
    for e in range(_HEADS):
        lo, hi = e * _HD, (e + 1) * _HD
        fe, ve = feat[lo:hi], val[lo:hi]
        ce, vce = cent[lo:hi], vcent[lo:hi]
        cn = jnp.sqrt(jnp.sum(ce * ce, axis=0, keepdims=True)) + eps
        pn = jnp.sqrt(jnp.sum(fe * fe, axis=0, keepdims=True)) + eps
        chat = (ce / cn).astype(bf16)
        fhat = (fe / pn).astype(bf16)
        cos = dot(chat, fhat, (((0,), (0,)), ((), ())))
        sim = jax.nn.sigmoid(beta + alpha * cos)
        simv = jnp.where(validm > 0.0, sim, -1.0)
        smax = jnp.max(simv, axis=0, keepdims=True)
        cand = jnp.where(simv == smax, ridx, 1e9)
        idx = jnp.min(cand, axis=0, keepdims=True)
        onehot = (ridx == idx).astype(f32)
        weight = onehot * smax
        smax2 = jnp.max(jnp.where(onehot > 0.0, -1.0, simv), axis=0,
                        keepdims=True)
        lc_acc += jnp.sum(smax)
        ls_acc += jnp.sum(smax2)
        aggT = hdot(ve, weight, (((1,), (1,)), ((), ())))
        cnt = hdot(ones_row, onehot, (((1,), (1,)), ((), ())))
        outcT = (aggT + vce) / (cnt + 1.0)
        disp_parts.append(hdot(outcT, weight, (((1,), (0,)), ((), ()))))
        cc = dot(chat, chat, (((0,), (0,)), ((), ())))
        lo_acc += jnp.sum(((cc - eye_ref[...]) ** 2) * blk_ref[...])
    disp = jnp.concatenate(disp_parts, axis=0).astype(bf16)
    out = dot(wp_ref[...].astype(bf16), disp,
              (((1,), (0,)), ((), ()))) + bp_ref[...]
    out_ref[0] = out
    lc_ref[...] = lc_ref[...] + lc_acc
    ls_ref[...] = ls_ref[...] + ls_acc
    lo_ref[...] = lo_ref[...] + lo_acc

    @pl.when(i == _B - 1)
    def _finalize():
        lc_ref[...] = -lc_ref[...] / _NPTS
        ls_ref[...] = ls_ref[...] / _NPTS
        lo_ref[...] = lo_ref[...] / _NORTH


def kernel(x, Wf, bf, Wv, bv, Wp, bp, sim_alpha, sim_beta):
    f32 = jnp.float32
    x16 = x.astype(jnp.bfloat16).reshape(_B, _C, _S)
    wfv = jnp.concatenate([Wf, Wv], axis=0)
    bfv = jnp.concatenate([bf, bv]).reshape(2 * _OC, 1)
    ab = jnp.concatenate([sim_alpha, sim_beta]).reshape(1, 2).astype(f32)

    full = lambda shape: pl.BlockSpec(shape, lambda i: (0,) * len(shape))
    out, lc, ls, lo = pl.pallas_call(
        _fec_kernel,
        grid=(_B,),
        in_specs=[
            pl.BlockSpec((1, _C, _S), lambda i: (i, 0, 0)),
            full((2 * _OC, _C)), full((2 * _OC, 1)),
            full((_C, _OC)), full((_C, 1)),
            full((1, 2)),
            full((_NJ, _S)), full((_NJ, _S)), full((_NJ, 1)),
            full((_NJ, _NJ)), full((_NJ, _NJ)),
        ],
        out_specs=[
            pl.BlockSpec((1, _C, _S), lambda i: (i, 0, 0)),
            full((1, 1)), full((1, 1)), full((1, 1)),
        ],
        out_shape=[
            jax.ShapeDtypeStruct((_B, _C, _S), f32),
            jax.ShapeDtypeStruct((1, 1), f32),
            jax.ShapeDtypeStruct((1, 1), f32),
            jax.ShapeDtypeStruct((1, 1), f32),
        ],
    )(x16, wfv, bfv, Wp, bp.reshape(_C, 1), ab,
      jnp.asarray(_POOL), jnp.asarray(_VALID), jnp.asarray(_RIDX),
      jnp.asarray(_EYE), jnp.asarray(_BLK))
    return out.reshape(_B, _C, _W0, _H0), lc[0, 0], ls[0, 0], lo[0, 0]


# speedup vs baseline: 1.5952x; 1.5952x over previous
import functools

import numpy as np
import jax
import jax.numpy as jnp
from jax.experimental import pallas as pl

_B, _C, _W0, _H0 = 8, 192, 56, 56
_HEADS, _HD = 4, 24
_OC = _HEADS * _HD
_S = _W0 * _H0
_NJ = 16
_NPTS = _B * _HEADS * _S
_NORTH = _B * _HEADS * 4 * 16


def _constants():
    s = np.arange(_S)
    w, h = s // _H0, s % _H0
    j = (w // 28) * 8 + (h // 28) * 4 + ((w // 14) % 2) * 2 + ((h // 14) % 2)
    rows = np.arange(_NJ)[:, None]
    pool = (rows == j[None, :]).astype(np.float32) / 196.0
    valid = ((rows // 4) == (j[None, :] // 4)).astype(np.float32)
    ridx = np.arange(_NJ, dtype=np.float32)[:, None]
    eye = np.eye(_NJ, dtype=np.float32)
    blk = ((np.arange(_NJ)[:, None] // 4) ==
           (np.arange(_NJ)[None, :] // 4)).astype(np.float32)
    return pool, valid, ridx, eye, blk


_POOL, _VALID, _RIDX, _EYE, _BLK = _constants()


def _fec_kernel(x_ref, wf_ref, bf_ref, wv_ref, bv_ref, wp_ref, bp_ref,
                ab_ref, pool_ref, valid_ref, ridx_ref, eye_ref, blk_ref,
                out_ref, lc_ref, ls_ref, lo_ref):
    i = pl.program_id(0)

    @pl.when(i == 0)
    def _init():
        lc_ref[...] = jnp.zeros((1, 1), jnp.float32)
        ls_ref[...] = jnp.zeros((1, 1), jnp.float32)
        lo_ref[...] = jnp.zeros((1, 1), jnp.float32)

    f32 = jnp.float32
    bf16 = jnp.bfloat16
    dot = functools.partial(jax.lax.dot_general, preferred_element_type=f32)
    hdot = functools.partial(jax.lax.dot_general, preferred_element_type=f32,
                             precision=jax.lax.Precision.HIGHEST)
    xb = x_ref[0].astype(bf16)
    feat = dot(wf_ref[...].astype(bf16), xb,
               (((1,), (0,)), ((), ()))) + bf_ref[...]
    val = dot(wv_ref[...].astype(bf16), xb,
              (((1,), (0,)), ((), ()))) + bv_ref[...]
    poolm = pool_ref[...]
    cent = hdot(feat, poolm, (((1,), (1,)), ((), ())))
    vcent = hdot(val, poolm, (((1,), (1,)), ((), ())))
    alpha = ab_ref[0, 0]
    beta = ab_ref[0, 1]
    validm = valid_ref[...]
    ridx = ridx_ref[...]
    eps = 1e-12
    ones_row = jnp.ones((1, _S), f32)
    disp_parts = []
    lc_acc = 0.0
    ls_acc = 0.0
    lo_acc = 0.0
    for e in range(_HEADS):
        lo, hi = e * _HD, (e + 1) * _HD
        fe, ve = feat[lo:hi], val[lo:hi]
        ce, vce = cent[lo:hi], vcent[lo:hi]
        cn = jnp.sqrt(jnp.sum(ce * ce, axis=0, keepdims=True)) + eps
        pn = jnp.sqrt(jnp.sum(fe * fe, axis=0, keepdims=True)) + eps
        chat = (ce / cn).astype(bf16)
        fhat = (fe / pn).astype(bf16)
        cos = dot(chat, fhat, (((0,), (0,)), ((), ())))
        sim = jax.nn.sigmoid(beta + alpha * cos)
        simv = jnp.where(validm > 0.0, sim, -1.0)
        smax = jnp.max(simv, axis=0, keepdims=True)
        cand = jnp.where(simv == smax, ridx, 1e9)
        idx = jnp.min(cand, axis=0, keepdims=True)
        onehot = (ridx == idx).astype(f32)
        weight = onehot * smax
        smax2 = jnp.max(jnp.where(onehot > 0.0, -1.0, simv), axis=0,
                        keepdims=True)
        lc_acc += jnp.sum(smax)
        ls_acc += jnp.sum(smax2)
        aggT = dot(ve, weight, (((1,), (1,)), ((), ())))
        cnt = dot(ones_row, onehot, (((1,), (1,)), ((), ())))
        outcT = (aggT + vce) / (cnt + 1.0)
        disp_parts.append(dot(outcT, weight, (((1,), (0,)), ((), ()))))
        cc = dot(chat, chat, (((0,), (0,)), ((), ())))
        lo_acc += jnp.sum(((cc - eye_ref[...]) ** 2) * blk_ref[...])
    disp = jnp.concatenate(disp_parts, axis=0).astype(bf16)
    out = dot(wp_ref[...].astype(bf16), disp,
              (((1,), (0,)), ((), ()))) + bp_ref[...]
    out_ref[0] = out
    lc_ref[...] = lc_ref[...] + lc_acc
    ls_ref[...] = ls_ref[...] + ls_acc
    lo_ref[...] = lo_ref[...] + lo_acc

    @pl.when(i == _B - 1)
    def _finalize():
        lc_ref[...] = -lc_ref[...] / _NPTS
        ls_ref[...] = ls_ref[...] / _NPTS
        lo_ref[...] = lo_ref[...] / _NORTH


def kernel(x, Wf, bf, Wv, bv, Wp, bp, sim_alpha, sim_beta):
    f32 = jnp.float32
    xf = x.reshape(_B, _C, _S)
    ab = jnp.concatenate([sim_alpha, sim_beta]).reshape(1, 2).astype(f32)

    full = lambda shape: pl.BlockSpec(shape, lambda i: (0,) * len(shape))
    out, lc, ls, lo = pl.pallas_call(
        _fec_kernel,
        grid=(_B,),
        in_specs=[
            pl.BlockSpec((1, _C, _S), lambda i: (i, 0, 0)),
            full((_OC, _C)), full((_OC, 1)),
            full((_OC, _C)), full((_OC, 1)),
            full((_C, _OC)), full((_C, 1)),
            full((1, 2)),
            full((_NJ, _S)), full((_NJ, _S)), full((_NJ, 1)),
            full((_NJ, _NJ)), full((_NJ, _NJ)),
        ],
        out_specs=[
            pl.BlockSpec((1, _C, _S), lambda i: (i, 0, 0)),
            full((1, 1)), full((1, 1)), full((1, 1)),
        ],
        out_shape=[
            jax.ShapeDtypeStruct((_B, _C, _S), f32),
            jax.ShapeDtypeStruct((1, 1), f32),
            jax.ShapeDtypeStruct((1, 1), f32),
            jax.ShapeDtypeStruct((1, 1), f32),
        ],
    )(xf, Wf, bf.reshape(_OC, 1), Wv, bv.reshape(_OC, 1),
      Wp, bp.reshape(_C, 1), ab, jnp.asarray(_POOL), jnp.asarray(_VALID),
      jnp.asarray(_RIDX), jnp.asarray(_EYE), jnp.asarray(_BLK))
    return out.reshape(_B, _C, _W0, _H0), lc[0, 0], ls[0, 0], lo[0, 0]


# speedup vs baseline: 1.7334x; 1.0867x over previous
import functools

import numpy as np
import jax
import jax.numpy as jnp
from jax.experimental import pallas as pl

_B, _C, _W0, _H0 = 8, 192, 56, 56
_HEADS, _HD = 4, 24
_OC = _HEADS * _HD
_S = _W0 * _H0
_NJ = 16
_NPTS = _B * _HEADS * _S
_NORTH = _B * _HEADS * 4 * 16


def _constants():
    s = np.arange(_S)
    w, h = s // _H0, s % _H0
    j = (w // 28) * 8 + (h // 28) * 4 + ((w // 14) % 2) * 2 + ((h // 14) % 2)
    rows = np.arange(_NJ)[:, None]
    pool = (rows == j[None, :]).astype(np.float32)
    valid = ((rows // 4) == (j[None, :] // 4)).astype(np.float32)
    ridx = np.arange(_NJ, dtype=np.float32)[:, None]
    eye = np.eye(_NJ, dtype=np.float32)
    blk = ((np.arange(_NJ)[:, None] // 4) ==
           (np.arange(_NJ)[None, :] // 4)).astype(np.float32)
    return pool, valid, ridx, eye, blk


_POOL, _VALID, _RIDX, _EYE, _BLK = _constants()


def _fec_kernel(x_ref, wf_ref, bf_ref, wv_ref, bv_ref, wp_ref, bp_ref,
                ab_ref, pool_ref, valid_ref, ridx_ref, eye_ref, blk_ref,
                out_ref, lc_ref, ls_ref, lo_ref):
    i = pl.program_id(0)

    @pl.when(i == 0)
    def _init():
        lc_ref[...] = jnp.zeros((1, 1), jnp.float32)
        ls_ref[...] = jnp.zeros((1, 1), jnp.float32)
        lo_ref[...] = jnp.zeros((1, 1), jnp.float32)

    f32 = jnp.float32
    bf16 = jnp.bfloat16
    dot = functools.partial(jax.lax.dot_general, preferred_element_type=f32)
    hdot = functools.partial(jax.lax.dot_general, preferred_element_type=f32,
                             precision=jax.lax.Precision.HIGHEST)
    xb = x_ref[0].astype(bf16)
    feat = dot(wf_ref[...].astype(bf16), xb,
               (((1,), (0,)), ((), ()))) + bf_ref[...]
    val = dot(wv_ref[...].astype(bf16), xb,
              (((1,), (0,)), ((), ()))) + bv_ref[...]
    poolm = pool_ref[...].astype(bf16)

    def _pool_mean(t):
        h = t.astype(bf16)
        r = t - h.astype(f32)
        m = r.astype(bf16)
        l = (r - m.astype(f32)).astype(bf16)
        acc = (dot(h, poolm, (((1,), (1,)), ((), ()))) +
               dot(m, poolm, (((1,), (1,)), ((), ()))) +
               dot(l, poolm, (((1,), (1,)), ((), ()))))
        return acc * (1.0 / 196.0)

    cent = _pool_mean(feat)
    vcent = _pool_mean(val)
    alpha = ab_ref[0, 0]
    beta = ab_ref[0, 1]
    validm = valid_ref[...]
    ridx = ridx_ref[...]
    eps = 1e-12
    ones_row = jnp.ones((1, _S), f32)
    disp_parts = []
    lc_acc = 0.0
    ls_acc = 0.0
    lo_acc = 0.0
    for e in range(_HEADS):
        lo, hi = e * _HD, (e + 1) * _HD
        fe, ve = feat[lo:hi], val[lo:hi]
        ce, vce = cent[lo:hi], vcent[lo:hi]
        cn = jnp.sqrt(jnp.sum(ce * ce, axis=0, keepdims=True)) + eps
        pn = jnp.sqrt(jnp.sum(fe * fe, axis=0, keepdims=True)) + eps
        chat = (ce / cn).astype(bf16)
        fhat = (fe / pn).astype(bf16)
        cos = dot(chat, fhat, (((0,), (0,)), ((), ())))
        sim = jax.nn.sigmoid(beta + alpha * cos)
        simv = jnp.where(validm > 0.0, sim, -1.0)
        smax = jnp.max(simv, axis=0, keepdims=True)
        cand = jnp.where(simv == smax, ridx, 1e9)
        idx = jnp.min(cand, axis=0, keepdims=True)
        onehot = (ridx == idx).astype(f32)
        weight = onehot * smax
        smax2 = jnp.max(jnp.where(onehot > 0.0, -1.0, simv), axis=0,
                        keepdims=True)
        lc_acc += jnp.sum(smax)
        ls_acc += jnp.sum(smax2)
        aggT = dot(ve, weight, (((1,), (1,)), ((), ())))
        cnt = dot(ones_row, onehot, (((1,), (1,)), ((), ())))
        outcT = (aggT + vce) / (cnt + 1.0)
        disp_parts.append(dot(outcT, weight, (((1,), (0,)), ((), ()))))
        cc = dot(chat, chat, (((0,), (0,)), ((), ())))
        lo_acc += jnp.sum(((cc - eye_ref[...]) ** 2) * blk_ref[...])
    disp = jnp.concatenate(disp_parts, axis=0).astype(bf16)
    out = dot(wp_ref[...].astype(bf16), disp,
              (((1,), (0,)), ((), ()))) + bp_ref[...]
    out_ref[0] = out
    lc_ref[...] = lc_ref[...] + lc_acc
    ls_ref[...] = ls_ref[...] + ls_acc
    lo_ref[...] = lo_ref[...] + lo_acc

    @pl.when(i == _B - 1)
    def _finalize():
        lc_ref[...] = -lc_ref[...] / _NPTS
        ls_ref[...] = ls_ref[...] / _NPTS
        lo_ref[...] = lo_ref[...] / _NORTH


def kernel(x, Wf, bf, Wv, bv, Wp, bp, sim_alpha, sim_beta):
    f32 = jnp.float32
    xf = x.reshape(_B, _C, _S)
    ab = jnp.concatenate([sim_alpha, sim_beta]).reshape(1, 2).astype(f32)

    full = lambda shape: pl.BlockSpec(shape, lambda i: (0,) * len(shape))
    out, lc, ls, lo = pl.pallas_call(
        _fec_kernel,
        grid=(_B,),
        in_specs=[
            pl.BlockSpec((1, _C, _S), lambda i: (i, 0, 0)),
            full((_OC, _C)), full((_OC, 1)),
            full((_OC, _C)), full((_OC, 1)),
            full((_C, _OC)), full((_C, 1)),
            full((1, 2)),
            full((_NJ, _S)), full((_NJ, _S)), full((_NJ, 1)),
            full((_NJ, _NJ)), full((_NJ, _NJ)),
        ],
        out_specs=[
            pl.BlockSpec((1, _C, _S), lambda i: (i, 0, 0)),
            full((1, 1)), full((1, 1)), full((1, 1)),
        ],
        out_shape=[
            jax.ShapeDtypeStruct((_B, _C, _S), f32),
            jax.ShapeDtypeStruct((1, 1), f32),
            jax.ShapeDtypeStruct((1, 1), f32),
            jax.ShapeDtypeStruct((1, 1), f32),
        ],
    )(xf, Wf, bf.reshape(_OC, 1), Wv, bv.reshape(_OC, 1),
      Wp, bp.reshape(_C, 1), ab, jnp.asarray(_POOL), jnp.asarray(_VALID),
      jnp.asarray(_RIDX), jnp.asarray(_EYE), jnp.asarray(_BLK))
    return out.reshape(_B, _C, _W0, _H0), lc[0, 0], ls[0, 0], lo[0, 0]
